# Initial kernel scaffold; baseline (speedup 1.0000x reference)
import jax
import jax.numpy as jnp
from jax.experimental import pallas as pl


def kernel(refined_sem, refined_ins, lseg_gt, mem_sem, mem_ins, mem_mask, inst_mask):
    raise NotImplementedError("write your pallas kernel here")


# speedup vs baseline: 3.9981x; 3.9981x over previous
import jax
import jax.numpy as jnp
from jax import lax
from jax.experimental import pallas as pl
from jax.experimental.pallas import tpu as pltpu

_F = 8
_N = 4096
_K = 16
_CS = 512
_CI = 64
_NB = 512
_NJ = _N // _NB
_EPS = 1e-12
_MARGIN = 0.2
_HI = lax.Precision.HIGHEST
_DN = (((1,), (1,)), ((), ()))


def _stage2_body(sem_ref, gt_ref, msem_ref, ins_ref, mins_ref, mask_ref,
                 ids_ref, out_ref, sp_ref, sg_ref, ft_ref, cnt_ref, sacc_ref):
    f = pl.program_id(0)
    j = pl.program_id(1)

    @pl.when((f == 0) & (j == 0))
    def _init():
        sp_ref[...] = jnp.zeros_like(sp_ref)
        sg_ref[...] = jnp.zeros_like(sg_ref)
        ft_ref[...] = jnp.zeros_like(ft_ref)
        cnt_ref[...] = jnp.zeros_like(cnt_ref)
        sacc_ref[0] = 0.0
        sacc_ref[1] = 0.0
        sacc_ref[2] = 0.0

    a = sem_ref[0]
    g = gt_ref[0]
    m = msem_ref[0]
    fi = ins_ref[0]
    mi = mins_ref[0]
    mm = mask_ref[0]
    ids = ids_ref[0]

    na = jnp.sum(a * a, axis=0, keepdims=True)
    ng = jnp.sum(g * g, axis=0, keepdims=True)
    nm = jnp.sum(m * m, axis=0, keepdims=True)
    dam = jnp.sum(a * m, axis=0, keepdims=True)
    inva = 1.0 / jnp.maximum(jnp.sqrt(na), _EPS)
    invg = 1.0 / jnp.maximum(jnp.sqrt(ng), _EPS)
    invm = 1.0 / jnp.maximum(jnp.sqrt(nm), _EPS)

    nfi = jnp.sum(fi * fi, axis=0, keepdims=True)
    nmi = jnp.sum(mi * mi, axis=0, keepdims=True)
    dfm = jnp.sum(fi * mi, axis=0, keepdims=True)
    invf = 1.0 / jnp.maximum(jnp.sqrt(nfi), _EPS)
    invmi = 1.0 / jnp.maximum(jnp.sqrt(nmi), _EPS)

    cos_sm = dam * inva * invm
    cos_im = dfm * invf * invmi
    sacc_ref[0] += jnp.sum(mm)
    sacc_ref[1] += jnp.sum((1.0 - cos_sm) * mm)
    sacc_ref[2] += jnp.sum((1.0 - cos_im) * mm)

    oh = (ids == lax.broadcasted_iota(jnp.int32, (_K, _NB), 0)).astype(jnp.float32)

    pred_n = a * inva
    gt_n = g * invg
    feat_n = fi * invf
    sp_ref[f] += lax.dot_general(pred_n, oh, _DN, precision=_HI,
                                 preferred_element_type=jnp.float32)
    sg_ref[f] += lax.dot_general(gt_n, oh, _DN, precision=_HI,
                                 preferred_element_type=jnp.float32)
    ft_ref[f] += lax.dot_general(feat_n, oh, _DN, precision=_HI,
                                 preferred_element_type=jnp.float32)
    cnt_ref[f] += lax.dot_general(jnp.ones((1, _NB), jnp.float32), oh, _DN,
                                  precision=_HI,
                                  preferred_element_type=jnp.float32)

    @pl.when((f == _F - 1) & (j == _NJ - 1))
    def _finalize():
        SP = sp_ref[...]
        SG = sg_ref[...]
        FT = ft_ref[...]
        cnt = cnt_ref[...]

        segk = lax.broadcasted_iota(jnp.int32, (_F, 1, _K), 2)
        fg = (segk > 0)

        ngp = jnp.sqrt(jnp.sum(SG * SG, axis=1, keepdims=True))
        dgp = jnp.sum(SG * SP, axis=1, keepdims=True)
        va = jnp.where(fg & (cnt >= 2.0), 1.0, 0.0)
        align_num = jnp.sum(va * (cnt - dgp / jnp.maximum(ngp, _EPS)))
        align_den = jnp.maximum(jnp.sum(va * cnt), 1.0)

        nf = jnp.sqrt(jnp.sum(FT * FT, axis=1, keepdims=True))
        vi = jnp.where(fg & (cnt >= 1.0), 1.0, 0.0)
        intra_num = jnp.sum(vi * (cnt - nf * nf / jnp.maximum(nf, _EPS)))
        intra_den = jnp.maximum(jnp.sum(vi * cnt), 1.0)

        pn = FT / jnp.maximum(nf, _EPS)
        sim = lax.dot_general(pn, pn, (((1,), (1,)), ((0,), (0,))),
                              precision=_HI,
                              preferred_element_type=jnp.float32)
        vv = lax.dot_general(vi, vi, (((1,), (1,)), ((0,), (0,))),
                             precision=_HI,
                             preferred_element_type=jnp.float32)
        r_i = lax.broadcasted_iota(jnp.int32, (_F, _K, _K), 1)
        c_i = lax.broadcasted_iota(jnp.int32, (_F, _K, _K), 2)
        pair = vv * jnp.where(r_i != c_i, 1.0, 0.0)
        inter_num = jnp.sum(jnp.maximum(sim - _MARGIN, 0.0) * pair)
        inter_den = jnp.maximum(jnp.sum(pair), 1.0)

        smm = jnp.maximum(sacc_ref[0], 1.0)
        obj = (0.5 * align_num / align_den
               + sacc_ref[1] / smm
               + intra_num / intra_den + inter_num / inter_den
               + sacc_ref[2] / smm)
        out_ref[...] = obj[None, None]


def kernel(refined_sem, refined_ins, lseg_gt, mem_sem, mem_ins, mem_mask,
           inst_mask):
    sem = refined_sem.reshape(_F, _CS, _N)
    gt = lseg_gt.reshape(_F, _CS, _N)
    msem = mem_sem.reshape(_F, _CS, _N)
    ins = refined_ins.reshape(_F, _CI, _N)
    mins = mem_ins.reshape(_F, _CI, _N)
    mask = mem_mask.reshape(_F * _NJ, 1, _NB)
    ids = inst_mask.astype(jnp.int32).reshape(_F * _NJ, 1, _NB)

    big_spec = pl.BlockSpec((1, _CS, _NB), lambda f, j: (f, 0, j))
    ins_spec = pl.BlockSpec((1, _CI, _NB), lambda f, j: (f, 0, j))
    row_spec = pl.BlockSpec((1, 1, _NB), lambda f, j: (f * _NJ + j, 0, 0))

    out = pl.pallas_call(
        _stage2_body,
        grid=(_F, _NJ),
        in_specs=[big_spec, big_spec, big_spec, ins_spec, ins_spec,
                  row_spec, row_spec],
        out_specs=pl.BlockSpec((1, 1), lambda f, j: (0, 0)),
        out_shape=jax.ShapeDtypeStruct((1, 1), jnp.float32),
        scratch_shapes=[
            pltpu.VMEM((_F, _CS, _K), jnp.float32),
            pltpu.VMEM((_F, _CS, _K), jnp.float32),
            pltpu.VMEM((_F, _CI, _K), jnp.float32),
            pltpu.VMEM((_F, 1, _K), jnp.float32),
            pltpu.SMEM((4,), jnp.float32),
        ],
    )(sem, gt, msem, ins, mins, mask, ids)
    return out[0, 0]


# speedup vs baseline: 4.7760x; 1.1946x over previous
import jax
import jax.numpy as jnp
from jax import lax
from jax.experimental import pallas as pl
from jax.experimental.pallas import tpu as pltpu

_F = 8
_N = 4096
_K = 16
_CS = 512
_CI = 64
_NB = 512
_NJ = _N // _NB
_EPS = 1e-12
_MARGIN = 0.2
_HI = lax.Precision.HIGHEST
_DN = (((1,), (1,)), ((), ()))
_DNB = (((2,), (2,)), ((0,), (0,)))


def _stage2_body(sem_ref, gt_ref, msem_ref, ins_ref, mins_ref, mask_ref,
                 ids_ref, out_ref, sp_ref, sg_ref, ft_ref, cnt_ref,
                 mm_ref, csm_ref, cim_ref):
    f = pl.program_id(0)
    j = pl.program_id(1)

    @pl.when((f == 0) & (j == 0))
    def _init():
        sp_ref[...] = jnp.zeros_like(sp_ref)
        sg_ref[...] = jnp.zeros_like(sg_ref)
        ft_ref[...] = jnp.zeros_like(ft_ref)
        cnt_ref[...] = jnp.zeros_like(cnt_ref)
        mm_ref[...] = jnp.zeros_like(mm_ref)
        csm_ref[...] = jnp.zeros_like(csm_ref)
        cim_ref[...] = jnp.zeros_like(cim_ref)

    a = sem_ref[0]
    g = gt_ref[0]
    m = msem_ref[0]
    fi = ins_ref[0]
    mi = mins_ref[0]
    mm = mask_ref[0]
    ids = ids_ref[0]

    na = jnp.sum(a * a, axis=0, keepdims=True)
    ng = jnp.sum(g * g, axis=0, keepdims=True)
    nm = jnp.sum(m * m, axis=0, keepdims=True)
    dam = jnp.sum(a * m, axis=0, keepdims=True)
    inva = 1.0 / jnp.maximum(jnp.sqrt(na), _EPS)
    invg = 1.0 / jnp.maximum(jnp.sqrt(ng), _EPS)
    invm = 1.0 / jnp.maximum(jnp.sqrt(nm), _EPS)

    nfi = jnp.sum(fi * fi, axis=0, keepdims=True)
    nmi = jnp.sum(mi * mi, axis=0, keepdims=True)
    dfm = jnp.sum(fi * mi, axis=0, keepdims=True)
    invf = 1.0 / jnp.maximum(jnp.sqrt(nfi), _EPS)
    invmi = 1.0 / jnp.maximum(jnp.sqrt(nmi), _EPS)

    mm_ref[...] += mm
    csm_ref[...] += (1.0 - dam * inva * invm) * mm
    cim_ref[...] += (1.0 - dfm * invf * invmi) * mm

    oh = (ids == lax.broadcasted_iota(jnp.int32, (_K, _NB), 0)).astype(jnp.float32)

    sp_ref[f] += lax.dot_general(oh * inva, a, _DN,
                                 preferred_element_type=jnp.float32)
    sg_ref[f] += lax.dot_general(oh * invg, g, _DN,
                                 preferred_element_type=jnp.float32)
    ft_ref[f] += lax.dot_general(oh * invf, fi, _DN,
                                 preferred_element_type=jnp.float32)
    cnt_ref[f] += jnp.sum(oh, axis=1, keepdims=True)

    @pl.when((f == _F - 1) & (j == _NJ - 1))
    def _finalize():
        SP = sp_ref[...]
        SG = sg_ref[...]
        FT = ft_ref[...]
        cnt = cnt_ref[...]

        segk = lax.broadcasted_iota(jnp.int32, (_F, _K, 1), 1)
        fg = (segk > 0)

        ngp = jnp.sqrt(jnp.sum(SG * SG, axis=2, keepdims=True))
        dgp = jnp.sum(SG * SP, axis=2, keepdims=True)
        va = jnp.where(fg & (cnt >= 2.0), 1.0, 0.0)
        align_num = jnp.sum(va * (cnt - dgp / jnp.maximum(ngp, _EPS)))
        align_den = jnp.maximum(jnp.sum(va * cnt), 1.0)

        nf = jnp.sqrt(jnp.sum(FT * FT, axis=2, keepdims=True))
        vi = jnp.where(fg & (cnt >= 1.0), 1.0, 0.0)
        intra_num = jnp.sum(vi * (cnt - nf * nf / jnp.maximum(nf, _EPS)))
        intra_den = jnp.maximum(jnp.sum(vi * cnt), 1.0)

        pn = FT / jnp.maximum(nf, _EPS)
        sim = lax.dot_general(pn, pn, _DNB, precision=_HI,
                              preferred_element_type=jnp.float32)
        vv = lax.dot_general(vi, vi, _DNB, precision=_HI,
                             preferred_element_type=jnp.float32)
        r_i = lax.broadcasted_iota(jnp.int32, (_F, _K, _K), 1)
        c_i = lax.broadcasted_iota(jnp.int32, (_F, _K, _K), 2)
        pair = vv * jnp.where(r_i != c_i, 1.0, 0.0)
        inter_num = jnp.sum(jnp.maximum(sim - _MARGIN, 0.0) * pair)
        inter_den = jnp.maximum(jnp.sum(pair), 1.0)

        smm = jnp.maximum(jnp.sum(mm_ref[...]), 1.0)
        obj = (0.5 * align_num / align_den
               + jnp.sum(csm_ref[...]) / smm
               + intra_num / intra_den + inter_num / inter_den
               + jnp.sum(cim_ref[...]) / smm)
        out_ref[...] = obj[None, None]


def kernel(refined_sem, refined_ins, lseg_gt, mem_sem, mem_ins, mem_mask,
           inst_mask):
    sem = refined_sem.reshape(_F, _CS, _N)
    gt = lseg_gt.reshape(_F, _CS, _N)
    msem = mem_sem.reshape(_F, _CS, _N)
    ins = refined_ins.reshape(_F, _CI, _N)
    mins = mem_ins.reshape(_F, _CI, _N)
    mask = mem_mask.reshape(_F * _NJ, 1, _NB)
    ids = inst_mask.astype(jnp.int32).reshape(_F * _NJ, 1, _NB)

    big_spec = pl.BlockSpec((1, _CS, _NB), lambda f, j: (f, 0, j))
    ins_spec = pl.BlockSpec((1, _CI, _NB), lambda f, j: (f, 0, j))
    row_spec = pl.BlockSpec((1, 1, _NB), lambda f, j: (f * _NJ + j, 0, 0))

    out = pl.pallas_call(
        _stage2_body,
        grid=(_F, _NJ),
        in_specs=[big_spec, big_spec, big_spec, ins_spec, ins_spec,
                  row_spec, row_spec],
        out_specs=pl.BlockSpec((1, 1), lambda f, j: (0, 0)),
        out_shape=jax.ShapeDtypeStruct((1, 1), jnp.float32),
        scratch_shapes=[
            pltpu.VMEM((_F, _K, _CS), jnp.float32),
            pltpu.VMEM((_F, _K, _CS), jnp.float32),
            pltpu.VMEM((_F, _K, _CI), jnp.float32),
            pltpu.VMEM((_F, _K, 1), jnp.float32),
            pltpu.VMEM((1, _NB), jnp.float32),
            pltpu.VMEM((1, _NB), jnp.float32),
            pltpu.VMEM((1, _NB), jnp.float32),
        ],
    )(sem, gt, msem, ins, mins, mask, ids)
    return out[0, 0]


# speedup vs baseline: 5.0351x; 1.0543x over previous
import jax
import jax.numpy as jnp
from jax import lax
from jax.experimental import pallas as pl
from jax.experimental.pallas import tpu as pltpu

_F = 8
_N = 4096
_K = 16
_CS = 512
_CI = 64
_NB = 4096
_NJ = _N // _NB
_EPS = 1e-12
_MARGIN = 0.2
_HI = lax.Precision.HIGHEST
_DN = (((1,), (1,)), ((), ()))
_DNB = (((2,), (2,)), ((0,), (0,)))


def _stage2_body(sem_ref, gt_ref, msem_ref, ins_ref, mins_ref, mask_ref,
                 ids_ref, out_ref, sp_ref, sg_ref, ft_ref, cnt_ref,
                 mm_ref, csm_ref, cim_ref):
    f = pl.program_id(0)
    j = pl.program_id(1)

    @pl.when((f == 0) & (j == 0))
    def _init():
        sp_ref[...] = jnp.zeros_like(sp_ref)
        sg_ref[...] = jnp.zeros_like(sg_ref)
        ft_ref[...] = jnp.zeros_like(ft_ref)
        cnt_ref[...] = jnp.zeros_like(cnt_ref)
        mm_ref[...] = jnp.zeros_like(mm_ref)
        csm_ref[...] = jnp.zeros_like(csm_ref)
        cim_ref[...] = jnp.zeros_like(cim_ref)

    a = sem_ref[0]
    g = gt_ref[0]
    m = msem_ref[0]
    fi = ins_ref[0]
    mi = mins_ref[0]
    mm = mask_ref[0]
    ids = ids_ref[0]

    na = jnp.sum(a * a, axis=0, keepdims=True)
    ng = jnp.sum(g * g, axis=0, keepdims=True)
    nm = jnp.sum(m * m, axis=0, keepdims=True)
    dam = jnp.sum(a * m, axis=0, keepdims=True)
    inva = 1.0 / jnp.maximum(jnp.sqrt(na), _EPS)
    invg = 1.0 / jnp.maximum(jnp.sqrt(ng), _EPS)
    invm = 1.0 / jnp.maximum(jnp.sqrt(nm), _EPS)

    nfi = jnp.sum(fi * fi, axis=0, keepdims=True)
    nmi = jnp.sum(mi * mi, axis=0, keepdims=True)
    dfm = jnp.sum(fi * mi, axis=0, keepdims=True)
    invf = 1.0 / jnp.maximum(jnp.sqrt(nfi), _EPS)
    invmi = 1.0 / jnp.maximum(jnp.sqrt(nmi), _EPS)

    mm_ref[...] += mm
    csm_ref[...] += (1.0 - dam * inva * invm) * mm
    cim_ref[...] += (1.0 - dfm * invf * invmi) * mm

    oh = (ids == lax.broadcasted_iota(jnp.int32, (_K, _NB), 0)).astype(jnp.float32)

    sp_ref[f] += lax.dot_general(oh * inva, a, _DN,
                                 preferred_element_type=jnp.float32)
    sg_ref[f] += lax.dot_general(oh * invg, g, _DN,
                                 preferred_element_type=jnp.float32)
    ft_ref[f] += lax.dot_general(oh * invf, fi, _DN,
                                 preferred_element_type=jnp.float32)
    cnt_ref[f] += jnp.sum(oh, axis=1, keepdims=True)

    @pl.when((f == _F - 1) & (j == _NJ - 1))
    def _finalize():
        SP = sp_ref[...]
        SG = sg_ref[...]
        FT = ft_ref[...]
        cnt = cnt_ref[...]

        segk = lax.broadcasted_iota(jnp.int32, (_F, _K, 1), 1)
        fg = (segk > 0)

        ngp = jnp.sqrt(jnp.sum(SG * SG, axis=2, keepdims=True))
        dgp = jnp.sum(SG * SP, axis=2, keepdims=True)
        va = jnp.where(fg & (cnt >= 2.0), 1.0, 0.0)
        align_num = jnp.sum(va * (cnt - dgp / jnp.maximum(ngp, _EPS)))
        align_den = jnp.maximum(jnp.sum(va * cnt), 1.0)

        nf = jnp.sqrt(jnp.sum(FT * FT, axis=2, keepdims=True))
        vi = jnp.where(fg & (cnt >= 1.0), 1.0, 0.0)
        intra_num = jnp.sum(vi * (cnt - nf * nf / jnp.maximum(nf, _EPS)))
        intra_den = jnp.maximum(jnp.sum(vi * cnt), 1.0)

        pn = FT / jnp.maximum(nf, _EPS)
        sim = lax.dot_general(pn, pn, _DNB, precision=_HI,
                              preferred_element_type=jnp.float32)
        vv = lax.dot_general(vi, vi, _DNB, precision=_HI,
                             preferred_element_type=jnp.float32)
        r_i = lax.broadcasted_iota(jnp.int32, (_F, _K, _K), 1)
        c_i = lax.broadcasted_iota(jnp.int32, (_F, _K, _K), 2)
        pair = vv * jnp.where(r_i != c_i, 1.0, 0.0)
        inter_num = jnp.sum(jnp.maximum(sim - _MARGIN, 0.0) * pair)
        inter_den = jnp.maximum(jnp.sum(pair), 1.0)

        smm = jnp.maximum(jnp.sum(mm_ref[...]), 1.0)
        obj = (0.5 * align_num / align_den
               + jnp.sum(csm_ref[...]) / smm
               + intra_num / intra_den + inter_num / inter_den
               + jnp.sum(cim_ref[...]) / smm)
        out_ref[...] = obj[None, None]


def kernel(refined_sem, refined_ins, lseg_gt, mem_sem, mem_ins, mem_mask,
           inst_mask):
    sem = refined_sem.reshape(_F, _CS, _N)
    gt = lseg_gt.reshape(_F, _CS, _N)
    msem = mem_sem.reshape(_F, _CS, _N)
    ins = refined_ins.reshape(_F, _CI, _N)
    mins = mem_ins.reshape(_F, _CI, _N)
    mask = mem_mask.reshape(_F * _NJ, 1, _NB)
    ids = inst_mask.astype(jnp.int32).reshape(_F * _NJ, 1, _NB)

    big_spec = pl.BlockSpec((1, _CS, _NB), lambda f, j: (f, 0, j))
    ins_spec = pl.BlockSpec((1, _CI, _NB), lambda f, j: (f, 0, j))
    row_spec = pl.BlockSpec((1, 1, _NB), lambda f, j: (f * _NJ + j, 0, 0))

    out = pl.pallas_call(
        _stage2_body,
        grid=(_F, _NJ),
        in_specs=[big_spec, big_spec, big_spec, ins_spec, ins_spec,
                  row_spec, row_spec],
        out_specs=pl.BlockSpec((1, 1), lambda f, j: (0, 0)),
        out_shape=jax.ShapeDtypeStruct((1, 1), jnp.float32),
        scratch_shapes=[
            pltpu.VMEM((_F, _K, _CS), jnp.float32),
            pltpu.VMEM((_F, _K, _CS), jnp.float32),
            pltpu.VMEM((_F, _K, _CI), jnp.float32),
            pltpu.VMEM((_F, _K, 1), jnp.float32),
            pltpu.VMEM((1, _NB), jnp.float32),
            pltpu.VMEM((1, _NB), jnp.float32),
            pltpu.VMEM((1, _NB), jnp.float32),
        ],
    )(sem, gt, msem, ins, mins, mask, ids)
    return out[0, 0]
